# baseline (speedup 1.0000x reference)
import functools

import jax
import jax.numpy as jnp
from jax.experimental import pallas as pl
from jax.experimental.pallas import tpu as pltpu

BLK = 2048


@functools.lru_cache(maxsize=None)
def _make_tc(n_idx: int, n_emb: int, d: int):
    nblk = n_idx // BLK

    def body(idx_ref, table_ref, out_ref):
        ids = idx_ref[0, 0, :]
        onehot = (ids[:, None] == jax.lax.iota(jnp.int32, n_emb)[None, :])
        out_ref[...] = jnp.dot(onehot.astype(jnp.float32), table_ref[...],
                               preferred_element_type=jnp.float32)

    return pl.pallas_call(
        body,
        grid=(nblk,),
        in_specs=[
            pl.BlockSpec((1, 1, BLK), lambda i: (i, 0, 0)),
            pl.BlockSpec((n_emb, d), lambda i: (0, 0)),
        ],
        out_specs=pl.BlockSpec((BLK, d), lambda i: (i, 0)),
        out_shape=jax.ShapeDtypeStruct((n_idx, d), jnp.float32),
    )


def kernel(idx, x, table):
    del x
    b, l = idx.shape
    n = b * l
    d = table.shape[1]
    idx3 = idx.reshape(n // BLK, 1, BLK).astype(jnp.int32)
    out = _make_tc(n, table.shape[0], d)(idx3, table.astype(jnp.float32))
    return out.reshape(b, l, d)
